# baseline (speedup 1.0000x reference)
import functools

import jax
import jax.numpy as jnp
from jax import lax
from jax.experimental import pallas as pl
from jax.experimental.pallas import tpu as pltpu
from jax.experimental.pallas import tpu_sc as plsc

N = 819200
D = 128
L = 16
NC, NS = 2, 16
NW = NC * NS
ROWS_PER_W = N // NW
CH = 128
CHUNKS = ROWS_PER_W // CH
VPR = D // L

_mesh = plsc.VectorSubcoreMesh(core_axis_name="c", subcore_axis_name="s")


@functools.partial(
    pl.kernel,
    mesh=_mesh,
    out_type=jax.ShapeDtypeStruct((N, D), jnp.float32),
    scratch_types=[
        pltpu.VMEM((2, 3, CH), jnp.int32),
        pltpu.VMEM((2, 3, CH, D), jnp.float32),
        pltpu.VMEM((8,), jnp.int32),
        pltpu.VMEM((8, D), jnp.float32),
        pltpu.SemaphoreType.DMA,
        pltpu.SemaphoreType.DMA,
        pltpu.SemaphoreType.DMA,
        pltpu.SemaphoreType.DMA,
        pltpu.SemaphoreType.DMA,
        pltpu.SemaphoreType.DMA,
    ],
)
def _sc_kernel(idxs_hbm, s_hbm, xt_hbm, yt_hbm, zt_hbm, st_hbm,
               out_hbm, idx, bufs, sidx, srow_v,
               isem0, isem1, gsem0, gsem1, osem0, osem1):
    isem = (isem0, isem1)
    gsem = (gsem0, gsem1)
    osem = (osem0, osem1)
    tabs = (xt_hbm, yt_hbm, zt_hbm)

    wid = lax.axis_index("s") * NC + lax.axis_index("c")
    base0 = wid * ROWS_PER_W

    pltpu.sync_copy(s_hbm, sidx)
    pltpu.async_copy(st_hbm.at[sidx], srow_v, gsem0).wait()
    srows = [srow_v[0, pl.ds(k * L, L)] for k in range(VPR)]

    def idx_desc(s, g):
        return pltpu.make_async_copy(
            idxs_hbm.at[:, pl.ds(base0 + g * CH, CH)], idx.at[s], isem[s])

    def gat_desc(s, t):
        return pltpu.make_async_copy(
            tabs[t].at[idx.at[s, t]], bufs.at[s, t], gsem[s])

    def out_desc(s, g):
        return pltpu.make_async_copy(
            bufs.at[s, 0], out_hbm.at[pl.ds(base0 + g * CH, CH), :], osem[s])

    idx_desc(0, 0).start()
    idx_desc(0, 0).wait()
    for t in range(3):
        gat_desc(0, t).start()
    idx_desc(1, 1).start()

    def pair_body(i, _):
        for s in (0, 1):
            g = 2 * i + s
            s2 = 1 - s

            @pl.when(g + 1 < CHUNKS)
            def _():
                idx_desc(s2, g + 1).wait()
                gat_desc(s2, 1).start()
                gat_desc(s2, 2).start()

            @pl.when(g > 0)
            def _():
                out_desc(s2, g).wait()

            @pl.when(g + 1 < CHUNKS)
            def _():
                gat_desc(s2, 0).start()

            for t in range(3):
                gat_desc(s, t).wait()

            @pl.when(g + 2 < CHUNKS)
            def _():
                idx_desc(s, g + 2).start()

            @plsc.parallel_loop(0, CH, unroll=4)
            def row_body(r):
                for k in range(VPR):
                    sl = pl.ds(k * L, L)
                    acc = bufs[s, 0, r, sl] + bufs[s, 1, r, sl]
                    acc = acc + bufs[s, 2, r, sl]
                    bufs[s, 0, r, sl] = acc + srows[k]

            out_desc(s, g).start()
        return 0

    lax.fori_loop(0, CHUNKS // 2, pair_body, 0)

    out_desc(1, CHUNKS - 1).wait()


def kernel(coords, stride, x_table, y_table, z_table, stride_table):
    idxs = coords[:, 1:4].T
    s_vec = jnp.full((8,), stride, dtype=jnp.int32)
    return _sc_kernel(idxs, s_vec, x_table, y_table, z_table, stride_table)
